# Initial kernel scaffold; baseline (speedup 1.0000x reference)
import jax
import jax.numpy as jnp
from jax.experimental import pallas as pl


def kernel(x, weight, bias, embedding):
    raise NotImplementedError("write your pallas kernel here")


# speedup vs baseline: 566.5642x; 566.5642x over previous
import functools

import jax
import jax.numpy as jnp
from jax import lax
from jax.experimental import pallas as pl

N = 256
S = 256
B = 8
K = 76

_HP = lax.Precision.HIGHEST
_DP = lax.Precision.DEFAULT


def _graph_kernel(x_ref, w_ref, bias_ref, emb_ref, embT_ref, out_ref):
    emb = emb_ref[...]
    embT = embT_ref[...]

    g = lax.dot_general(emb, emb, (((1,), (1,)), ((), ())),
                        precision=_DP, preferred_element_type=jnp.float32)
    nsq_col = jnp.sum(emb * emb, axis=1, keepdims=True)
    nsq_row = jnp.sum(embT * embT, axis=0, keepdims=True)
    cos = g / (jnp.sqrt(nsq_col) * jnp.sqrt(nsq_row) + 1e-8)

    bits = lax.bitcast_convert_type(cos, jnp.uint32)
    signbit = jnp.uint32(0x80000000)
    uk = jnp.where(bits >= signbit, ~bits, bits + signbit)

    def _bisect(t, kacc):
        bit = jnp.uint32(31) - jnp.uint32(t)
        trial = kacc | (jnp.uint32(1) << bit)
        cnt = jnp.sum((uk >= trial).astype(jnp.int32), axis=0, keepdims=True)
        return jnp.where(cnt >= K, trial, kacc)

    kth = lax.fori_loop(0, 32, _bisect, jnp.zeros((1, N), jnp.uint32))

    gt = uk > kth
    eq = uk == kth
    g_cnt = jnp.sum(gt.astype(jnp.int32), axis=0, keepdims=True)
    need = (K - g_cnt).astype(jnp.float32)
    eqf = eq.astype(jnp.float32)
    cc = eqf
    for sh in (1, 2, 4, 8, 16, 32, 64, 128):
        cc = cc + jnp.concatenate(
            [jnp.zeros((sh, N), jnp.float32), cc[: N - sh, :]], axis=0)
    cc = cc - eqf
    mt = jnp.where(gt | (eq & (cc < need)), 1.0, 0.0)

    eyef = (lax.broadcasted_iota(jnp.int32, (N, N), 0)
            == lax.broadcasted_iota(jnp.int32, (N, N), 1)).astype(jnp.float32)
    m = lax.dot_general(mt, eyef, (((0,), (0,)), ((), ())),
                        precision=_HP, preferred_element_type=jnp.float32)
    adj = jnp.where(mt + m > 0, 1.0, 0.0)
    nbr = jnp.maximum(adj, eyef)
    common = lax.dot_general(nbr, nbr, (((1,), (1,)), ((), ())),
                             precision=_HP, preferred_element_type=jnp.float32)
    maxc = jnp.max(jnp.max(common, axis=1, keepdims=True), axis=0,
                   keepdims=True)
    coeff = jnp.where((adj > 0) & (common > 1), (common / maxc) * common, 0.0)

    a = mt * coeff
    deg = jnp.sum(a, axis=0, keepdims=True)
    dinv = jnp.where(deg > 0, lax.rsqrt(deg), 0.0)

    w = w_ref[...]
    bias = bias_ref[...]
    for b in range(B):
        xb = x_ref[b]
        h = lax.dot_general(w, xb, (((0,), (0,)), ((), ())),
                            precision=_DP, preferred_element_type=jnp.float32)
        o = lax.dot_general(h * dinv, a, (((1,), (0,)), ((), ())),
                            precision=_HP, preferred_element_type=jnp.float32)
        out_ref[b] = o * dinv + bias


@jax.jit
def kernel(x, weight, bias, embedding):
    out = pl.pallas_call(
        _graph_kernel,
        out_shape=jax.ShapeDtypeStruct((B, S, N), jnp.float32),
    )(x, weight, bias[:, None], embedding, embedding.T)
    return out


# speedup vs baseline: 604.6969x; 1.0673x over previous
import functools

import jax
import jax.numpy as jnp
from jax import lax
from jax.experimental import pallas as pl

N = 256
S = 256
B = 8
K = 76

_HP = lax.Precision.HIGHEST
_DP = lax.Precision.DEFAULT


def _graph_kernel(x_ref, wt_ref, bias_ref, emb_ref, embT_ref, out_ref):
    emb = emb_ref[...]
    embT = embT_ref[...]

    g = lax.dot_general(emb, emb, (((1,), (1,)), ((), ())),
                        precision=_DP, preferred_element_type=jnp.float32)
    nsq_col = jnp.sum(emb * emb, axis=1, keepdims=True)
    nsq_row = jnp.sum(embT * embT, axis=0, keepdims=True)
    cos = g / (jnp.sqrt(nsq_col) * jnp.sqrt(nsq_row) + 1e-8)

    bits = lax.bitcast_convert_type(cos, jnp.uint32)
    signbit = jnp.uint32(0x80000000)
    uk = jnp.where(bits >= signbit, ~bits, bits + signbit)

    def _bisect(t, kacc):
        bit = jnp.uint32(31) - jnp.uint32(t)
        trial = kacc | (jnp.uint32(1) << bit)
        cnt = jnp.sum((uk >= trial).astype(jnp.int32), axis=0, keepdims=True)
        return jnp.where(cnt >= K, trial, kacc)

    kth = lax.fori_loop(0, 32, _bisect, jnp.zeros((1, N), jnp.uint32))

    gt = uk > kth
    eq = uk == kth
    g_cnt = jnp.sum(gt.astype(jnp.int32), axis=0, keepdims=True)
    need = (K - g_cnt).astype(jnp.float32)
    eqf = eq.astype(jnp.float32)
    cc = eqf
    for sh in (1, 2, 4, 8, 16, 32, 64, 128):
        cc = cc + jnp.concatenate(
            [jnp.zeros((sh, N), jnp.float32), cc[: N - sh, :]], axis=0)
    cc = cc - eqf
    mt = jnp.where(gt | (eq & (cc < need)), 1.0, 0.0)

    eyef = (lax.broadcasted_iota(jnp.int32, (N, N), 0)
            == lax.broadcasted_iota(jnp.int32, (N, N), 1)).astype(jnp.float32)
    m = lax.dot_general(mt, eyef, (((0,), (0,)), ((), ())),
                        precision=_DP, preferred_element_type=jnp.float32)
    adj = jnp.where(mt + m > 0, 1.0, 0.0)
    nbr = jnp.maximum(adj, eyef)
    common = lax.dot_general(nbr, nbr, (((1,), (1,)), ((), ())),
                             precision=_DP, preferred_element_type=jnp.float32)
    maxc = jnp.max(jnp.max(common, axis=1, keepdims=True), axis=0,
                   keepdims=True)
    coeff = jnp.where((adj > 0) & (common > 1), (common / maxc) * common, 0.0)

    a = mt * coeff
    deg = jnp.sum(a, axis=0, keepdims=True)
    dinv = jnp.where(deg > 0, lax.rsqrt(deg), 0.0)

    wt = wt_ref[...]
    bias = bias_ref[...]
    for b in range(B):
        xb = x_ref[b]
        h = lax.dot_general(wt, xb, (((1,), (0,)), ((), ())),
                            precision=_DP, preferred_element_type=jnp.float32)
        o = lax.dot_general(h * dinv, a, (((1,), (0,)), ((), ())),
                            precision=_DP, preferred_element_type=jnp.float32)
        out_ref[b] = o * dinv + bias


@jax.jit
def kernel(x, weight, bias, embedding):
    out = pl.pallas_call(
        _graph_kernel,
        out_shape=jax.ShapeDtypeStruct((B, S, N), jnp.float32),
    )(x, weight.T, bias[:, None], embedding, embedding.T)
    return out
